# baseline (speedup 1.0000x reference)
import jax
import jax.numpy as jnp
from jax.experimental import pallas as pl
from jax.experimental.pallas import tpu as pltpu

HIDDEN = 2048
NUM_EXPERTS = 16
TOP_K = 2
BETA = 0.1
GAMMA = 0.1
EXPLORATION_C = 0.1
LOAD_EMA_ALPHA = 0.9

TB = 1024
E = NUM_EXPERTS


def _logits_block(x_ref, w_ref, bias_ref, out_ref):
    out_ref[...] = jax.lax.dot_general(
        w_ref[...], x_ref[...],
        dimension_numbers=(((1,), (1,)), ((), ())),
        preferred_element_type=jnp.float32) + bias_ref[:, 0:1]


def _tc_logits(x2, W, bias2):
    T = x2.shape[0]
    return pl.pallas_call(
        _logits_block,
        grid=(T // TB,),
        in_specs=[
            pl.BlockSpec((TB, HIDDEN), lambda i: (i, 0)),
            pl.BlockSpec((E, HIDDEN), lambda i: (0, 0)),
            pl.BlockSpec((E, 128), lambda i: (0, 0)),
        ],
        out_specs=pl.BlockSpec((E, TB), lambda i: (0, i)),
        out_shape=jax.ShapeDtypeStruct((E, T), jnp.float32),
    )(x2, W, bias2)


def _route_block(lg_ref, wout_ref, iout_ref, aux_ref, acc_ref):
    step = pl.program_id(0)
    nsteps = pl.num_programs(0)

    @pl.when(step == 0)
    def _():
        acc_ref[...] = jnp.zeros_like(acc_ref)

    lgT = lg_ref[...]
    iota_e = jax.lax.broadcasted_iota(jnp.int32, (E, TB), 0)

    m1 = jnp.max(lgT, axis=0, keepdims=True)
    i1 = jnp.min(jnp.where(lgT == m1, iota_e, E), axis=0, keepdims=True)
    masked = jnp.where(iota_e == i1, -jnp.inf, lgT)
    m2 = jnp.max(masked, axis=0, keepdims=True)
    i2 = jnp.min(jnp.where(masked == m2, iota_e, E), axis=0, keepdims=True)

    e2 = jnp.exp(m2 - m1)
    w1 = 1.0 / (1.0 + e2)
    w2 = 1.0 - w1

    wout_ref[...] = jnp.concatenate([w1, w2], axis=0).T.reshape(1, TB, 2)
    iout_ref[...] = jnp.concatenate([i1, i2], axis=0).T.reshape(1, TB, 2)

    p = jnp.exp(lgT - m1)
    probs = p / jnp.sum(p, axis=0, keepdims=True)
    prob_sum = jnp.sum(probs, axis=1, keepdims=True)
    gate = ((iota_e == i1).astype(jnp.float32)
            + (iota_e == i2).astype(jnp.float32))
    cnt_sum = jnp.sum(gate, axis=1, keepdims=True)

    acc_ref[:, 0:1] += prob_sum
    acc_ref[:, 1:2] += cnt_sum

    @pl.when(step == nsteps - 1)
    def _():
        total_t = jnp.float32(TB) * nsteps
        aux = (jnp.sum(acc_ref[:, 0:1] * acc_ref[:, 1:2])
               * NUM_EXPERTS / (total_t * total_t))
        aux_ref[0, 0] = aux


def _tc_route(lgT):
    T = lgT.shape[1]
    return pl.pallas_call(
        _route_block,
        grid=(T // TB,),
        in_specs=[pl.BlockSpec((E, TB), lambda i: (0, i))],
        out_specs=[
            pl.BlockSpec((1, TB, TOP_K), lambda i: (i, 0, 0)),
            pl.BlockSpec((1, TB, TOP_K), lambda i: (i, 0, 0)),
            pl.BlockSpec(memory_space=pltpu.SMEM),
        ],
        out_shape=[
            jax.ShapeDtypeStruct((T // TB, TB, TOP_K), jnp.float32),
            jax.ShapeDtypeStruct((T // TB, TB, TOP_K), jnp.int32),
            jax.ShapeDtypeStruct((1, 1), jnp.float32),
        ],
        scratch_shapes=[pltpu.VMEM((E, 128), jnp.float32)],
    )(lgT)


def kernel(x, W, reputation_scores, expert_loads, expert_counts,
           total_routing_decisions):
    B, S, H = x.shape
    x2 = x.reshape(-1, H)
    updated_loads = (LOAD_EMA_ALPHA * expert_loads
                     + (1.0 - LOAD_EMA_ALPHA) * expert_loads)
    exploration = EXPLORATION_C * jnp.sqrt(
        jnp.log(total_routing_decisions + 1.0) / (expert_counts + 1e-10))
    bias = (BETA * reputation_scores - GAMMA * updated_loads
            + exploration).astype(jnp.float32)
    bias2 = jnp.broadcast_to(bias.reshape(E, 1), (E, 128))

    lgT = _tc_logits(x2, W, bias2)
    wout, iout, aux = _tc_route(lgT)
    routing_weights = wout.reshape(B, S, TOP_K)
    expert_indices = iout.reshape(B, S, TOP_K)
    return routing_weights, expert_indices, aux.reshape(())


def _unused_sc_note():
    pass


# speedup vs baseline: 1.1287x; 1.1287x over previous
import jax
import jax.numpy as jnp
from jax.experimental import pallas as pl
from jax.experimental.pallas import tpu as pltpu

HIDDEN = 2048
NUM_EXPERTS = 16
TOP_K = 2
BETA = 0.1
GAMMA = 0.1
EXPLORATION_C = 0.1
LOAD_EMA_ALPHA = 0.9

TB = 1024
E = NUM_EXPERTS


def _router_block(x_ref, w_ref, bias_ref, wout_ref, iout_ref, aux_ref,
                  acc_ref):
    step = pl.program_id(0)
    nsteps = pl.num_programs(0)

    @pl.when(step == 0)
    def _():
        acc_ref[...] = jnp.zeros_like(acc_ref)

    lgT = jax.lax.dot_general(
        w_ref[...], x_ref[...],
        dimension_numbers=(((1,), (1,)), ((), ())),
        preferred_element_type=jnp.float32) + bias_ref[:, 0:1]

    iota_e = jax.lax.broadcasted_iota(jnp.int32, (E, TB), 0)

    m1 = jnp.max(lgT, axis=0, keepdims=True)
    i1 = jnp.min(jnp.where(lgT == m1, iota_e, E), axis=0, keepdims=True)
    masked = jnp.where(iota_e == i1, -jnp.inf, lgT)
    m2 = jnp.max(masked, axis=0, keepdims=True)
    i2 = jnp.min(jnp.where(masked == m2, iota_e, E), axis=0, keepdims=True)

    e2 = jnp.exp(m2 - m1)
    w1 = 1.0 / (1.0 + e2)
    w2 = 1.0 - w1

    wout_ref[...] = jnp.concatenate([w1, w2], axis=0).T
    iout_ref[...] = jnp.concatenate([i1, i2], axis=0).T

    p = jnp.exp(lgT - m1)
    probs = p / jnp.sum(p, axis=0, keepdims=True)
    prob_sum = jnp.sum(probs, axis=1, keepdims=True)
    gate = ((iota_e == i1).astype(jnp.float32)
            + (iota_e == i2).astype(jnp.float32))
    cnt_sum = jnp.sum(gate, axis=1, keepdims=True)

    acc_ref[:, 0:1] += prob_sum
    acc_ref[:, 1:2] += cnt_sum

    @pl.when(step == nsteps - 1)
    def _():
        total_t = jnp.float32(TB) * nsteps
        aux = (jnp.sum(acc_ref[:, 0:1] * acc_ref[:, 1:2])
               * NUM_EXPERTS / (total_t * total_t))
        aux_ref[0, 0] = aux


def _router(x2, W, bias2):
    T = x2.shape[0]
    grid = (T // TB,)
    wout, iout, aux = pl.pallas_call(
        _router_block,
        grid=grid,
        in_specs=[
            pl.BlockSpec((TB, HIDDEN), lambda i: (i, 0)),
            pl.BlockSpec((E, HIDDEN), lambda i: (0, 0)),
            pl.BlockSpec((E, 128), lambda i: (0, 0)),
        ],
        out_specs=[
            pl.BlockSpec((TB, TOP_K), lambda i: (i, 0)),
            pl.BlockSpec((TB, TOP_K), lambda i: (i, 0)),
            pl.BlockSpec(memory_space=pltpu.SMEM),
        ],
        out_shape=[
            jax.ShapeDtypeStruct((T, TOP_K), jnp.float32),
            jax.ShapeDtypeStruct((T, TOP_K), jnp.int32),
            jax.ShapeDtypeStruct((1, 1), jnp.float32),
        ],
        scratch_shapes=[pltpu.VMEM((E, 128), jnp.float32)],
    )(x2, W, bias2)
    return wout, iout, aux


def kernel(x, W, reputation_scores, expert_loads, expert_counts,
           total_routing_decisions):
    B, S, H = x.shape
    x2 = x.reshape(-1, H)
    updated_loads = (LOAD_EMA_ALPHA * expert_loads
                     + (1.0 - LOAD_EMA_ALPHA) * expert_loads)
    exploration = EXPLORATION_C * jnp.sqrt(
        jnp.log(total_routing_decisions + 1.0) / (expert_counts + 1e-10))
    bias = (BETA * reputation_scores - GAMMA * updated_loads
            + exploration).astype(jnp.float32)
    bias2 = jnp.broadcast_to(bias.reshape(E, 1), (E, 128))

    wout, iout, aux = _router(x2, W, bias2)
    routing_weights = wout.reshape(B, S, TOP_K)
    expert_indices = iout.reshape(B, S, TOP_K)
    return routing_weights, expert_indices, aux.reshape(())
